# baseline (speedup 1.0000x reference)
import functools

import jax
import jax.numpy as jnp
from jax import lax
from jax.experimental import pallas as pl
from jax.experimental.pallas import tpu as pltpu
from jax.experimental.pallas import tpu_sc as plsc

DMODEL = 1024
DQK = 64
DV = 64
HEADS = 16
FF = 2048
NB = 64
PEN = 100000.0
D = 2
N = 4096
H = DMODEL // 2
CH = 128
NBLK = D * N // CH
NCHUNK = N // CH


def _sortpos_body(p_ref, pos_ref):
    p1 = p_ref[...]
    p = jnp.concatenate([p1, -p1], axis=-1)
    i64 = lax.broadcasted_iota(jnp.int32, (D * N, NB), 1).astype(jnp.float32)
    m = jnp.max(p, axis=-1, keepdims=True)
    hash_col = jnp.min(jnp.where(p == m, i64, float(NB)), axis=-1, keepdims=True)
    oh = (hash_col == i64).astype(jnp.float32)

    r = lax.broadcasted_iota(jnp.int32, (CH, CH), 0)
    c = lax.broadcasted_iota(jnp.int32, (CH, CH), 1)
    tri = (c <= r).astype(jnp.float32)

    pieces = []
    for b in range(D):
        base_row = jnp.zeros((1, NB), jnp.float32)
        ranks, bases, ohs = [], [], []
        for ck in range(NCHUNK):
            lo = b * N + ck * CH
            oh_c = oh[lo:lo + CH, :]
            cum_c = jnp.dot(tri, oh_c, precision=lax.Precision.HIGHEST)
            ranks.append(jnp.sum(oh_c * cum_c, axis=-1, keepdims=True))
            bases.append(jnp.sum(oh_c * base_row, axis=-1, keepdims=True))
            ohs.append(oh_c)
            base_row = base_row + cum_c[CH - 1:CH, :]
        br = lax.broadcasted_iota(jnp.int32, (NB, NB), 0)
        bc = lax.broadcasted_iota(jnp.int32, (NB, NB), 1)
        excl = (br < bc).astype(jnp.float32)
        start_row = jnp.dot(base_row, excl, precision=lax.Precision.HIGHEST)
        for ck in range(NCHUNK):
            st = jnp.sum(ohs[ck] * start_row, axis=-1, keepdims=True)
            pieces.append(ranks[ck] + bases[ck] + st - 1.0 + float(b * N))
    pos = jnp.concatenate(pieces, axis=0)
    pos_ref[...] = pos.astype(jnp.int32)


def _sortpos(pfull):
    return pl.pallas_call(
        _sortpos_body,
        in_specs=[pl.BlockSpec((D * N, NB // 2), lambda: (0, 0))],
        out_specs=pl.BlockSpec((D * N, 1), lambda: (0, 0)),
        out_shape=jax.ShapeDtypeStruct((D * N, 1), jnp.int32),
    )(pfull)


def _scatter_body(x2_hbm, pos_hbm, out_hbm, idx_v, rows0, rows1, sem0, sem1):
    cid = lax.axis_index("c")
    sid = lax.axis_index("s")
    wid = sid * 2 + cid
    pltpu.sync_copy(pos_hbm.at[wid], idx_v)
    bufs = (rows0, rows1)
    sems = (sem0, sem1)
    cps = [None, None, None, None]
    for j in range(4):
        if j >= 2:
            cps[j - 2].wait()
        pltpu.sync_copy(
            x2_hbm.at[pl.ds(wid * 256 + j * 64, 64), pl.ds(H, H)], bufs[j % 2])
        cps[j] = pltpu.async_copy(bufs[j % 2], out_hbm.at[idx_v.at[j]], sems[j % 2])
    cps[2].wait()
    cps[3].wait()


@functools.partial(
    pl.kernel,
    mesh=plsc.VectorSubcoreMesh(core_axis_name="c", subcore_axis_name="s"),
    out_type=jax.ShapeDtypeStruct((D * N, H), jnp.float32),
    scratch_types=[
        pltpu.VMEM((4, 64), jnp.int32),
        pltpu.VMEM((64, H), jnp.float32),
        pltpu.VMEM((64, H), jnp.float32),
        pltpu.SemaphoreType.DMA,
        pltpu.SemaphoreType.DMA,
    ],
)
def _scatter_rows(x2_flat, pos3, out, idx_v, rows0, rows1, sem0, sem1):
    _scatter_body(x2_flat, pos3, out, idx_v, rows0, rows1, sem0, sem1)


def _ln_rows(x, g, b):
    mu = jnp.mean(x, axis=-1, keepdims=True)
    xc = x - mu
    var = jnp.mean(xc * xc, axis=-1, keepdims=True)
    return xc * lax.rsqrt(var + 1e-5) * g + b


def _mm(a, b, dims=None):
    if dims is None:
        dims = (((a.ndim - 1,), (0,)), ((), ()))
    return lax.dot_general(a.astype(jnp.bfloat16), b.astype(jnp.bfloat16),
                           dims, preferred_element_type=jnp.float32)


ROWS = 1024
NSTEP = D * N // ROWS


def _main_body(x_ref, x2s_ref, wqkv_ref, bvu_ref, wb_ref, bb_ref, bu_ref,
               n1g_ref, n1b_ref, w1_ref, b1_ref, w2_ref, b2_ref, n2g_ref,
               n2b_ref, out_ref):
    xb = x_ref[0]
    x1 = xb[:, :H]
    x2 = xb[:, H:]
    xs = x2s_ref[...]
    xs16 = xs.astype(jnp.bfloat16)
    qv = _mm(xs16, wqkv_ref[...])
    t = qv[:, :H].astype(jnp.bfloat16)
    vfu = (qv[:, H:] + bvu_ref[...]).astype(jnp.bfloat16)
    cprow = _mm(wb_ref[...], xs16, (((1,), (1,)), ((), ())))
    bb = bb_ref[0, 0]
    r = lax.broadcasted_iota(jnp.int32, (CH, CH), 0)
    c = lax.broadcasted_iota(jnp.int32, (CH, CH), 1)
    dscale = jnp.where(r == c, 1.0 / (DQK ** 0.5) / PEN, 1.0 / (DQK ** 0.5))
    attns = []
    for j in range(ROWS // CH):
        sl = slice(j * CH, (j + 1) * CH)
        sj = _mm(t[sl, :], xs16[sl, :], (((1,), (1,)), ((), ())))
        cpj = cprow[:, sl]
        sj = sj + cpj + jnp.transpose(cpj) + bb
        sj = (sj * dscale).astype(jnp.bfloat16)
        attns.append(_mm(sj, vfu[sl, :]))
    uattn = jnp.concatenate(attns, axis=0) + bu_ref[...]
    y1 = _ln_rows(x1 + uattn, n1g_ref[...], n1b_ref[...])
    ffm = jnp.maximum(_mm(y1, w1_ref[...]) + b1_ref[...], 0.0).astype(jnp.bfloat16)
    ffo = _mm(ffm, w2_ref[...]) + b2_ref[...]
    y2 = _ln_rows(x2 + ffo, n2g_ref[...], n2b_ref[...])
    out_ref[0] = jnp.concatenate([y1, y2], axis=-1)


def _full(shape):
    return pl.BlockSpec(shape, lambda i: tuple(0 for _ in shape))


def _main(x, x2s, Wqk_w, Wqk_b, Wv_w, Wv_b, unify_w, unify_b, n1_g, n1_b,
          ff_w1, ff_b1, ff_w2, ff_b2, n2_g, n2_b):
    nrow = N // ROWS
    gram = Wqk_w @ Wqk_w.T
    wvu = Wv_w @ unify_w
    bvu = (Wv_b @ unify_w).reshape(1, -1)
    wqkv = jnp.concatenate([gram, wvu], axis=1)
    wb = (Wqk_w @ Wqk_b).reshape(1, -1)
    bb = (Wqk_b @ Wqk_b).reshape(1, 1)
    return pl.pallas_call(
        _main_body,
        grid=(NSTEP,),
        in_specs=[
            pl.BlockSpec((1, ROWS, DMODEL), lambda i: (i // nrow, i % nrow, 0)),
            pl.BlockSpec((ROWS, H), lambda i: (i, 0)),
            _full((H, 2 * H)),
            _full((1, H)),
            _full((1, H)),
            _full((1, 1)),
            _full((1, H)),
            _full((1, H)),
            _full((1, H)),
            _full((H, FF)),
            _full((1, FF)),
            _full((FF, H)),
            _full((1, H)),
            _full((1, H)),
            _full((1, H)),
        ],
        out_specs=pl.BlockSpec((1, ROWS, DMODEL), lambda i: (i // nrow, i % nrow, 0)),
        out_shape=jax.ShapeDtypeStruct((D, N, DMODEL), jnp.float32),
    )(x, x2s, wqkv, bvu, wb, bb,
      unify_b.reshape(1, -1), n1_g.reshape(1, -1), n1_b.reshape(1, -1),
      ff_w1, ff_b1.reshape(1, -1), ff_w2, ff_b2.reshape(1, -1),
      n2_g.reshape(1, -1), n2_b.reshape(1, -1))


def kernel(x, Wqk_w, Wqk_b, Wv_w, Wv_b, unify_w, unify_b, n1_g, n1_b,
           ff_w1, ff_b1, ff_w2, ff_b2, n2_g, n2_b, hashM):
    x2 = x[:, :, H:]
    qk = x2 @ Wqk_w + Wqk_b
    p1 = qk @ hashM
    pos = _sortpos(p1.reshape(D * N, NB // 2))
    pos3 = pos.reshape(32, 4, 64)
    x2s = _scatter_rows(x.reshape(D * N, DMODEL), pos3)
    return _main(x, x2s, Wqk_w, Wqk_b, Wv_w, Wv_b, unify_w, unify_b,
                 n1_g, n1_b, ff_w1, ff_b1, ff_w2, ff_b2, n2_g, n2_b)


# speedup vs baseline: 1.0095x; 1.0095x over previous
import functools

import jax
import jax.numpy as jnp
from jax import lax
from jax.experimental import pallas as pl
from jax.experimental.pallas import tpu as pltpu
from jax.experimental.pallas import tpu_sc as plsc

DMODEL = 1024
DQK = 64
DV = 64
HEADS = 16
FF = 2048
NB = 64
PEN = 100000.0
D = 2
N = 4096
H = DMODEL // 2
CH = 128
NBLK = D * N // CH
NCHUNK = N // CH


def _sortpos_body(p_ref, pos_ref):
    p1 = p_ref[0]
    p = jnp.concatenate([p1, -p1], axis=-1)
    i64 = lax.broadcasted_iota(jnp.int32, (N, NB), 1).astype(jnp.float32)
    m = jnp.max(p, axis=-1, keepdims=True)
    hash_col = jnp.min(jnp.where(p == m, i64, float(NB)), axis=-1, keepdims=True)
    oh = (hash_col == i64).astype(jnp.float32)

    r = lax.broadcasted_iota(jnp.int32, (CH, CH), 0)
    c = lax.broadcasted_iota(jnp.int32, (CH, CH), 1)
    tri = (c <= r).astype(jnp.float32)

    base_row = jnp.zeros((1, NB), jnp.float32)
    ranks, bases, ohs = [], [], []
    for ck in range(NCHUNK):
        oh_c = oh[ck * CH:(ck + 1) * CH, :]
        cum_c = jnp.dot(tri, oh_c, precision=lax.Precision.HIGHEST)
        ranks.append(jnp.sum(oh_c * cum_c, axis=-1, keepdims=True))
        bases.append(jnp.sum(oh_c * base_row, axis=-1, keepdims=True))
        ohs.append(oh_c)
        base_row = base_row + cum_c[CH - 1:CH, :]
    br = lax.broadcasted_iota(jnp.int32, (NB, NB), 0)
    bc = lax.broadcasted_iota(jnp.int32, (NB, NB), 1)
    excl = (br < bc).astype(jnp.float32)
    start_row = jnp.dot(base_row, excl, precision=lax.Precision.HIGHEST)

    pieces = []
    for ck in range(NCHUNK):
        st = jnp.sum(ohs[ck] * start_row, axis=-1, keepdims=True)
        pieces.append(ranks[ck] + bases[ck] + st - 1.0)
    pos = jnp.concatenate(pieces, axis=0)
    pos_ref[0] = pos.astype(jnp.int32) + pl.program_id(0) * N


def _sortpos(pfull):
    return pl.pallas_call(
        _sortpos_body,
        grid=(D,),
        in_specs=[pl.BlockSpec((1, N, NB // 2), lambda b: (b, 0, 0))],
        out_specs=pl.BlockSpec((1, N, 1), lambda b: (b, 0, 0)),
        out_shape=jax.ShapeDtypeStruct((D, N, 1), jnp.int32),
    )(pfull)


def _scatter_body(x2_hbm, pos_hbm, out_hbm, idx_v, rows0, rows1, rows2,
                  rsem0, rsem1, rsem2, sem0, sem1, sem2):
    cid = lax.axis_index("c")
    sid = lax.axis_index("s")
    wid = sid * 2 + cid
    bufs = (rows0, rows1, rows2)
    rsems = (rsem0, rsem1, rsem2)
    ssems = (sem0, sem1, sem2)
    pltpu.sync_copy(pos_hbm.at[wid], idx_v)
    rds = [None] * 4
    scs = [None] * 4
    for j in range(3):
        rds[j] = pltpu.async_copy(
            x2_hbm.at[pl.ds(wid * 256 + j * 64, 64), pl.ds(H, H)],
            bufs[j], rsems[j])
    for j in range(4):
        if j == 3:
            scs[0].wait()
            rds[3] = pltpu.async_copy(
                x2_hbm.at[pl.ds(wid * 256 + 3 * 64, 64), pl.ds(H, H)],
                bufs[0], rsems[0])
        rds[j].wait()
        scs[j] = pltpu.async_copy(bufs[j % 3], out_hbm.at[idx_v.at[j]],
                                  ssems[j % 3])
    scs[1].wait()
    scs[2].wait()
    scs[3].wait()


@functools.partial(
    pl.kernel,
    mesh=plsc.VectorSubcoreMesh(core_axis_name="c", subcore_axis_name="s"),
    out_type=jax.ShapeDtypeStruct((D * N, H), jnp.float32),
    scratch_types=[
        pltpu.VMEM((4, 64), jnp.int32),
        pltpu.VMEM((64, H), jnp.float32),
        pltpu.VMEM((64, H), jnp.float32),
        pltpu.VMEM((64, H), jnp.float32),
        pltpu.SemaphoreType.DMA,
        pltpu.SemaphoreType.DMA,
        pltpu.SemaphoreType.DMA,
        pltpu.SemaphoreType.DMA,
        pltpu.SemaphoreType.DMA,
        pltpu.SemaphoreType.DMA,
    ],
)
def _scatter_rows(x2_flat, pos3, out, idx_v, rows0, rows1, rows2,
                  rsem0, rsem1, rsem2, sem0, sem1, sem2):
    _scatter_body(x2_flat, pos3, out, idx_v, rows0, rows1, rows2,
                  rsem0, rsem1, rsem2, sem0, sem1, sem2)


def _ln_rows(x, g, b):
    mu = jnp.mean(x, axis=-1, keepdims=True)
    xc = x - mu
    var = jnp.mean(xc * xc, axis=-1, keepdims=True)
    return xc * lax.rsqrt(var + 1e-5) * g + b


def _mm(a, b, dims=None):
    if dims is None:
        dims = (((a.ndim - 1,), (0,)), ((), ()))
    return lax.dot_general(a.astype(jnp.bfloat16), b.astype(jnp.bfloat16),
                           dims, preferred_element_type=jnp.float32)


ROWS = 1024
NSTEP = D * N // ROWS


def _main_body(x_ref, x2s_ref, wqkv_ref, bvu_ref, wb_ref, bb_ref, bu_ref,
               n1g_ref, n1b_ref, w1_ref, b1_ref, w2_ref, b2_ref, n2g_ref,
               n2b_ref, out_ref):
    xb = x_ref[0]
    x1 = xb[:, :H]
    x2 = xb[:, H:]
    xs = x2s_ref[...]
    xs16 = xs.astype(jnp.bfloat16)
    qv = _mm(xs16, wqkv_ref[...])
    t = qv[:, :H].astype(jnp.bfloat16)
    vfu = (qv[:, H:] + bvu_ref[...]).astype(jnp.bfloat16)
    cprow = _mm(wb_ref[...], xs16, (((1,), (1,)), ((), ())))
    bb = bb_ref[0, 0]
    r = lax.broadcasted_iota(jnp.int32, (CH, CH), 0)
    c = lax.broadcasted_iota(jnp.int32, (CH, CH), 1)
    dscale = jnp.where(r == c, 1.0 / (DQK ** 0.5) / PEN, 1.0 / (DQK ** 0.5))
    attns = []
    for j in range(ROWS // CH):
        sl = slice(j * CH, (j + 1) * CH)
        sj = _mm(t[sl, :], xs16[sl, :], (((1,), (1,)), ((), ())))
        cpj = cprow[:, sl]
        sj = sj + cpj + jnp.transpose(cpj) + bb
        sj = (sj * dscale).astype(jnp.bfloat16)
        attns.append(_mm(sj, vfu[sl, :]))
    uattn = jnp.concatenate(attns, axis=0) + bu_ref[...]
    y1 = _ln_rows(x1 + uattn, n1g_ref[...], n1b_ref[...])
    ffm = jnp.maximum(_mm(y1, w1_ref[...]) + b1_ref[...], 0.0).astype(jnp.bfloat16)
    ffo = _mm(ffm, w2_ref[...]) + b2_ref[...]
    y2 = _ln_rows(x2 + ffo, n2g_ref[...], n2b_ref[...])
    out_ref[0] = jnp.concatenate([y1, y2], axis=-1)


def _full(shape):
    return pl.BlockSpec(shape, lambda i: tuple(0 for _ in shape))


def _main(x, x2s, Wqk_w, Wqk_b, Wv_w, Wv_b, unify_w, unify_b, n1_g, n1_b,
          ff_w1, ff_b1, ff_w2, ff_b2, n2_g, n2_b):
    nrow = N // ROWS
    gram = Wqk_w @ Wqk_w.T
    wvu = Wv_w @ unify_w
    bvu = (Wv_b @ unify_w).reshape(1, -1)
    wqkv = jnp.concatenate([gram, wvu], axis=1)
    wb = (Wqk_w @ Wqk_b).reshape(1, -1)
    bb = (Wqk_b @ Wqk_b).reshape(1, 1)
    return pl.pallas_call(
        _main_body,
        grid=(NSTEP,),
        in_specs=[
            pl.BlockSpec((1, ROWS, DMODEL), lambda i: (i // nrow, i % nrow, 0)),
            pl.BlockSpec((ROWS, H), lambda i: (i, 0)),
            _full((H, 2 * H)),
            _full((1, H)),
            _full((1, H)),
            _full((1, 1)),
            _full((1, H)),
            _full((1, H)),
            _full((1, H)),
            _full((H, FF)),
            _full((1, FF)),
            _full((FF, H)),
            _full((1, H)),
            _full((1, H)),
            _full((1, H)),
        ],
        out_specs=pl.BlockSpec((1, ROWS, DMODEL), lambda i: (i // nrow, i % nrow, 0)),
        out_shape=jax.ShapeDtypeStruct((D, N, DMODEL), jnp.float32),
    )(x, x2s, wqkv, bvu, wb, bb,
      unify_b.reshape(1, -1), n1_g.reshape(1, -1), n1_b.reshape(1, -1),
      ff_w1, ff_b1.reshape(1, -1), ff_w2, ff_b2.reshape(1, -1),
      n2_g.reshape(1, -1), n2_b.reshape(1, -1))


def kernel(x, Wqk_w, Wqk_b, Wv_w, Wv_b, unify_w, unify_b, n1_g, n1_b,
           ff_w1, ff_b1, ff_w2, ff_b2, n2_g, n2_b, hashM):
    x2 = x[:, :, H:]
    qk = x2 @ Wqk_w + Wqk_b
    p1 = qk @ hashM
    pos = _sortpos(p1)
    pos3 = pos.reshape(32, 4, 64)
    x2s = _scatter_rows(x.reshape(D * N, DMODEL), pos3)
    return _main(x, x2s, Wqk_w, Wqk_b, Wv_w, Wv_b, unify_w, unify_b,
                 n1_g, n1_b, ff_w1, ff_b1, ff_w2, ff_b2, n2_g, n2_b)
